# baseline (speedup 1.0000x reference)
import functools

import jax
import jax.numpy as jnp
from jax.experimental import pallas as pl
from jax.experimental.pallas import tpu as pltpu

_E = 64
_K = 8


def _router_kernel(x_ref, w_ref, ts_ref, ti_ref, loss_ref, acc_ref, *,
                   num_tokens, num_blocks):
    i = pl.program_id(0)

    @pl.when(i == 0)
    def _init():
        acc_ref[...] = jnp.zeros_like(acc_ref)

    scores = jax.lax.dot_general(
        w_ref[...], x_ref[...].astype(jnp.bfloat16),
        dimension_numbers=(((1,), (1,)), ((), ())),
        preferred_element_type=jnp.float32,
    )

    colmax = jnp.max(scores, axis=0, keepdims=True)
    e = jnp.exp(scores - colmax)
    denom = jnp.sum(e, axis=0, keepdims=True)
    gates = e / denom

    subl = jax.lax.broadcasted_iota(jnp.int32, scores.shape, 0).astype(jnp.float32)
    s = scores
    topmask = jnp.zeros_like(scores)
    vals = []
    idxs = []
    for _ in range(_K):
        m = jnp.max(s, axis=0, keepdims=True)
        hit = s == m
        idx = jnp.min(jnp.where(hit, subl, float(_E)), axis=0, keepdims=True)
        sel = subl == idx
        topmask = topmask + sel.astype(jnp.float32)
        s = jnp.where(sel, -jnp.inf, s)
        vals.append(jnp.exp(m - colmax))
        idxs.append(idx)

    ev = jnp.concatenate(vals, axis=0)
    tsum = jnp.sum(ev, axis=0, keepdims=True)
    ts_ref[...] = (ev / tsum).T
    ti_ref[...] = jnp.concatenate(idxs, axis=0).astype(jnp.int32).T

    acc_ref[:, 0:1] += jnp.sum(gates, axis=1, keepdims=True)
    acc_ref[:, 1:2] += jnp.sum(topmask, axis=1, keepdims=True)

    @pl.when(i == num_blocks - 1)
    def _finish():
        inv_n = 1.0 / num_tokens
        prod = (acc_ref[:, 0:1] * inv_n) * (acc_ref[:, 1:2] * inv_n)
        loss_ref[...] = _E * jnp.sum(prod, axis=0, keepdims=True)


def kernel(hidden_states, W):
    B, S, H = hidden_states.shape
    N = B * S
    x = hidden_states.reshape(N, H)
    TM = 1024 if N % 1024 == 0 else N
    num_blocks = N // TM

    body = functools.partial(_router_kernel, num_tokens=N, num_blocks=num_blocks)
    ts_t, ti_t, loss = pl.pallas_call(
        body,
        grid=(num_blocks,),
        in_specs=[
            pl.BlockSpec((TM, H), lambda i: (i, 0)),
            pl.BlockSpec((_E, H), lambda i: (0, 0)),
        ],
        out_specs=[
            pl.BlockSpec((TM, _K), lambda i: (i, 0)),
            pl.BlockSpec((TM, _K), lambda i: (i, 0)),
            pl.BlockSpec((1, 1), lambda i: (0, 0)),
        ],
        out_shape=[
            jax.ShapeDtypeStruct((N, _K), jnp.float32),
            jax.ShapeDtypeStruct((N, _K), jnp.int32),
            jax.ShapeDtypeStruct((1, 1), jnp.float32),
        ],
        scratch_shapes=[pltpu.VMEM((_E, 2), jnp.float32)],
    )(x, W.astype(jnp.bfloat16))
    return ts_t, ti_t, loss[0, 0]


# speedup vs baseline: 1.1845x; 1.1845x over previous
import functools

import jax
import jax.numpy as jnp
from jax.experimental import pallas as pl
from jax.experimental.pallas import tpu as pltpu

_E = 64
_K = 8


def _router_kernel(xa_ref, xb_ref, w_ref, ts_ref, ti_ref, loss_ref, acc_ref, *,
                   num_tokens, num_blocks):
    i = pl.program_id(0)

    @pl.when(i == 0)
    def _init():
        acc_ref[...] = jnp.zeros_like(acc_ref)

    sa = jax.lax.dot_general(
        w_ref[...], xa_ref[...].astype(jnp.bfloat16),
        dimension_numbers=(((1,), (1,)), ((), ())),
        preferred_element_type=jnp.float32,
    )
    sb = jax.lax.dot_general(
        w_ref[...], xb_ref[...].astype(jnp.bfloat16),
        dimension_numbers=(((1,), (1,)), ((), ())),
        preferred_element_type=jnp.float32,
    )
    scores = jnp.concatenate([sa, sb], axis=1)

    colmax = jnp.max(scores, axis=0, keepdims=True)
    e = jnp.exp(scores - colmax)
    denom = jnp.sum(e, axis=0, keepdims=True)
    gates = e / denom

    subl = jax.lax.broadcasted_iota(jnp.int32, scores.shape, 0).astype(jnp.float32)
    s = scores
    topmask = jnp.zeros_like(scores)
    vals = []
    idxs = []
    for _ in range(_K):
        m = jnp.max(s, axis=0, keepdims=True)
        hit = s == m
        idx = jnp.min(jnp.where(hit, subl, float(_E)), axis=0, keepdims=True)
        sel = subl == idx
        topmask = topmask + sel.astype(jnp.float32)
        s = jnp.where(sel, -jnp.inf, s)
        vals.append(jnp.exp(m - colmax))
        idxs.append(idx)

    ev = jnp.concatenate(vals, axis=0)
    tsum = jnp.sum(ev, axis=0, keepdims=True)
    ts_ref[...] = ev / tsum
    ti_ref[...] = jnp.concatenate(idxs, axis=0).astype(jnp.int32)

    acc_ref[:, 0:1] += jnp.sum(gates, axis=1, keepdims=True)
    acc_ref[:, 1:2] += jnp.sum(topmask, axis=1, keepdims=True)

    @pl.when(i == num_blocks - 1)
    def _finish():
        inv_n = 1.0 / num_tokens
        prod = (acc_ref[:, 0:1] * inv_n) * (acc_ref[:, 1:2] * inv_n)
        loss_ref[...] = _E * jnp.sum(prod, axis=0, keepdims=True)


def kernel(hidden_states, W):
    B, S, H = hidden_states.shape
    N = B * S
    x = hidden_states.reshape(N, H)
    TM = 1024 if N % 1024 == 0 else N
    num_blocks = N // TM

    body = functools.partial(_router_kernel, num_tokens=N, num_blocks=num_blocks)
    ts_t, ti_t, loss = pl.pallas_call(
        body,
        grid=(num_blocks,),
        in_specs=[
            pl.BlockSpec((TM // 2, H), lambda i: (2 * i, 0)),
            pl.BlockSpec((TM // 2, H), lambda i: (2 * i + 1, 0)),
            pl.BlockSpec((_E, H), lambda i: (0, 0)),
        ],
        out_specs=[
            pl.BlockSpec((_K, TM), lambda i: (0, i)),
            pl.BlockSpec((_K, TM), lambda i: (0, i)),
            pl.BlockSpec((1, 1), lambda i: (0, 0)),
        ],
        out_shape=[
            jax.ShapeDtypeStruct((_K, N), jnp.float32),
            jax.ShapeDtypeStruct((_K, N), jnp.int32),
            jax.ShapeDtypeStruct((1, 1), jnp.float32),
        ],
        scratch_shapes=[pltpu.VMEM((_E, 2), jnp.float32)],
    )(x, x, W.astype(jnp.bfloat16))
    return ts_t.T, ti_t.T, loss[0, 0]


# speedup vs baseline: 1.1864x; 1.0016x over previous
import functools

import jax
import jax.numpy as jnp
from jax.experimental import pallas as pl
from jax.experimental.pallas import tpu as pltpu

_E = 64
_K = 8


def _router_kernel(x_ref, w_ref, ts_ref, ti_ref, loss_ref, acc_ref, *,
                   num_tokens, num_blocks):
    i = pl.program_id(0)

    @pl.when(i == 0)
    def _init():
        acc_ref[...] = jnp.zeros_like(acc_ref)

    scores = jax.lax.dot_general(
        w_ref[...], x_ref[...].astype(jnp.bfloat16),
        dimension_numbers=(((1,), (1,)), ((), ())),
        preferred_element_type=jnp.float32,
    )

    colmax = jnp.max(scores, axis=0, keepdims=True)
    e = jnp.exp(scores - colmax)
    denom = jnp.sum(e, axis=0, keepdims=True)
    gates = e / denom

    subl = jax.lax.broadcasted_iota(jnp.int32, scores.shape, 0).astype(jnp.float32)
    s = scores
    topmask = jnp.zeros_like(scores)
    vals = []
    idxs = []
    for _ in range(_K):
        m = jnp.max(s, axis=0, keepdims=True)
        hit = s == m
        idx = jnp.min(jnp.where(hit, subl, float(_E)), axis=0, keepdims=True)
        sel = subl == idx
        topmask = topmask + sel.astype(jnp.float32)
        s = jnp.where(sel, -jnp.inf, s)
        vals.append(jnp.exp(m - colmax))
        idxs.append(idx)

    ev = jnp.concatenate(vals, axis=0)
    tsum = jnp.sum(ev, axis=0, keepdims=True)
    ts_ref[...] = ev / tsum
    ti_ref[...] = jnp.concatenate(idxs, axis=0).astype(jnp.int32)

    acc_ref[:, 0:1] += jnp.sum(gates, axis=1, keepdims=True)
    acc_ref[:, 1:2] += jnp.sum(topmask, axis=1, keepdims=True)

    @pl.when(i == num_blocks - 1)
    def _finish():
        inv_n = 1.0 / num_tokens
        prod = (acc_ref[:, 0:1] * inv_n) * (acc_ref[:, 1:2] * inv_n)
        loss_ref[...] = _E * jnp.sum(prod, axis=0, keepdims=True)


def kernel(hidden_states, W):
    B, S, H = hidden_states.shape
    N = B * S
    x = hidden_states.reshape(N, H)
    TM = 1024 if N % 1024 == 0 else N
    num_blocks = N // TM

    body = functools.partial(_router_kernel, num_tokens=N, num_blocks=num_blocks)
    ts_t, ti_t, loss = pl.pallas_call(
        body,
        grid=(num_blocks,),
        in_specs=[
            pl.BlockSpec((TM, H), lambda i: (i, 0)),
            pl.BlockSpec((_E, H), lambda i: (0, 0)),
        ],
        out_specs=[
            pl.BlockSpec((_K, TM), lambda i: (0, i)),
            pl.BlockSpec((_K, TM), lambda i: (0, i)),
            pl.BlockSpec((1, 1), lambda i: (0, 0)),
        ],
        out_shape=[
            jax.ShapeDtypeStruct((_K, N), jnp.float32),
            jax.ShapeDtypeStruct((_K, N), jnp.int32),
            jax.ShapeDtypeStruct((1, 1), jnp.float32),
        ],
        scratch_shapes=[pltpu.VMEM((_E, 2), jnp.float32)],
    )(x, W.astype(jnp.bfloat16))
    return ts_t.T, ti_t.T, loss[0, 0]
